# baseline (speedup 1.0000x reference)
import jax
import jax.numpy as jnp
from jax import lax
from jax.experimental import pallas as pl
from jax.experimental.pallas import tpu as pltpu
from jax.experimental.pallas import tpu_sc as plsc

BATCH = 16384
EMBED_DIM = 2
NUM_WORKERS = 16
BPW = BATCH // NUM_WORKERS
OPW = BPW * EMBED_DIM
L = 16
NCHUNK = 4
CHUNK = BPW // NCHUNK


def _sc_body(idx_hbm, tab_hbm, out_hbm, idx_v, tab_v, out_v, sems):
    s = lax.axis_index("s")
    base = s * BPW
    sem_t, sem_i, sem_o = sems
    cp_tab = pltpu.async_copy(tab_hbm, tab_v, sem_t)
    cp_in = [
        pltpu.async_copy(
            idx_hbm.at[pl.ds(base + q * CHUNK, CHUNK)],
            idx_v.at[pl.ds(q * CHUNK, CHUNK)],
            sem_i[q],
        )
        for q in range(NCHUNK)
    ]

    iota = lax.iota(jnp.int32, L)
    two_iota = iota * 2
    cp_tab.wait()
    t00 = tab_v[0, :]
    t01 = tab_v[1, :]
    t10 = tab_v[2, :]
    t11 = tab_v[3, :]
    t20 = tab_v[4, :]
    t21 = tab_v[5, :]

    cp_out = []
    for q in range(NCHUNK):
        cp_in[q].wait()
        for k in range(q * (CHUNK // L), (q + 1) * (CHUNK // L)):
            idx16 = idx_v[pl.ds(k * L, L)]
            w0 = (1 - jnp.minimum(idx16, 1)).astype(jnp.float32)
            w2 = jnp.maximum(idx16 - 1, 0).astype(jnp.float32)
            w1 = 1.0 - w0 - w2
            g0 = t00 * w0 + t10 * w1 + t20 * w2
            g1 = t01 * w0 + t11 * w1 + t21 * w2
            plsc.store_scatter(out_v, [two_iota + k * 2 * L], g0)
            plsc.store_scatter(out_v, [two_iota + (k * 2 * L + 1)], g1)
        cp_out.append(
            pltpu.async_copy(
                out_v.at[pl.ds(q * CHUNK * EMBED_DIM, CHUNK * EMBED_DIM)],
                out_hbm.at[
                    pl.ds(
                        base * EMBED_DIM + q * CHUNK * EMBED_DIM,
                        CHUNK * EMBED_DIM,
                    )
                ],
                sem_o[q],
            )
        )
    for cp in cp_out:
        cp.wait()


def kernel(x, emb_table):
    xi = x.astype(jnp.int32)
    tab_rep = jnp.tile(emb_table.reshape(3 * EMBED_DIM, 1), (1, L))
    mesh = plsc.VectorSubcoreMesh(
        core_axis_name="c", subcore_axis_name="s", num_cores=1
    )
    out_flat = pl.kernel(
        _sc_body,
        out_type=jax.ShapeDtypeStruct((BATCH * EMBED_DIM,), jnp.float32),
        mesh=mesh,
        compiler_params=pltpu.CompilerParams(needs_layout_passes=False),
        scratch_types=[
            pltpu.VMEM((BPW,), jnp.int32),
            pltpu.VMEM((3 * EMBED_DIM, L), jnp.float32),
            pltpu.VMEM((OPW,), jnp.float32),
            (
                pltpu.SemaphoreType.DMA,
                [pltpu.SemaphoreType.DMA] * NCHUNK,
                [pltpu.SemaphoreType.DMA] * NCHUNK,
            ),
        ],
    )(xi, tab_rep)
    return out_flat.reshape(BATCH, EMBED_DIM)


# speedup vs baseline: 1.0456x; 1.0456x over previous
import jax
import jax.numpy as jnp
from jax import lax
from jax.experimental import pallas as pl
from jax.experimental.pallas import tpu as pltpu
from jax.experimental.pallas import tpu_sc as plsc

BATCH = 16384
EMBED_DIM = 2
NUM_WORKERS = 16
BPW = BATCH // NUM_WORKERS
OPW = BPW * EMBED_DIM
L = 16
NCHUNK = 4
CHUNK = BPW // NCHUNK


def _sc_body(idx_hbm, tab_hbm, out_hbm, idx_v, tab_v, out_v, sems):
    s = lax.axis_index("s")
    base = s * BPW
    sem_t, sem_i, sem_o = sems
    cp_tab = pltpu.async_copy(tab_hbm, tab_v, sem_t)
    cp_in = [
        pltpu.async_copy(
            idx_hbm.at[pl.ds(base + q * CHUNK, CHUNK)],
            idx_v.at[pl.ds(q * CHUNK, CHUNK)],
            sem_i[q],
        )
        for q in range(NCHUNK)
    ]

    iota = lax.iota(jnp.int32, L)
    two_iota = iota * 2
    cp_tab.wait()
    t00 = tab_v[0, :]
    t01 = tab_v[1, :]
    t10 = tab_v[2, :]
    t11 = tab_v[3, :]
    t20 = tab_v[4, :]
    t21 = tab_v[5, :]

    cp_out = []
    for q in range(NCHUNK):
        cp_in[q].wait()
        for k in range(q * (CHUNK // L), (q + 1) * (CHUNK // L)):
            idx16 = idx_v[pl.ds(k * L, L)]
            is0 = idx16 == 0
            is1 = idx16 == 1
            g0 = jnp.where(is0, t00, jnp.where(is1, t10, t20))
            g1 = jnp.where(is0, t01, jnp.where(is1, t11, t21))
            plsc.store_scatter(out_v, [two_iota + k * 2 * L], g0)
            plsc.store_scatter(out_v, [two_iota + (k * 2 * L + 1)], g1)
        cp_out.append(
            pltpu.async_copy(
                out_v.at[pl.ds(q * CHUNK * EMBED_DIM, CHUNK * EMBED_DIM)],
                out_hbm.at[
                    pl.ds(
                        base * EMBED_DIM + q * CHUNK * EMBED_DIM,
                        CHUNK * EMBED_DIM,
                    )
                ],
                sem_o[q],
            )
        )
    for cp in cp_out:
        cp.wait()


def kernel(x, emb_table):
    xi = x.astype(jnp.int32)
    tab_rep = jnp.tile(emb_table.reshape(3 * EMBED_DIM, 1), (1, L))
    mesh = plsc.VectorSubcoreMesh(
        core_axis_name="c", subcore_axis_name="s", num_cores=1
    )
    out_flat = pl.kernel(
        _sc_body,
        out_type=jax.ShapeDtypeStruct((BATCH * EMBED_DIM,), jnp.float32),
        mesh=mesh,
        compiler_params=pltpu.CompilerParams(needs_layout_passes=False),
        scratch_types=[
            pltpu.VMEM((BPW,), jnp.int32),
            pltpu.VMEM((3 * EMBED_DIM, L), jnp.float32),
            pltpu.VMEM((OPW,), jnp.float32),
            (
                pltpu.SemaphoreType.DMA,
                [pltpu.SemaphoreType.DMA] * NCHUNK,
                [pltpu.SemaphoreType.DMA] * NCHUNK,
            ),
        ],
    )(xi, tab_rep)
    return out_flat.reshape(BATCH, EMBED_DIM)
